# baseline (speedup 1.0000x reference)
import functools

import jax
import jax.numpy as jnp
from jax import lax
from jax.experimental import pallas as pl
from jax.experimental.pallas import tpu as pltpu
from jax.experimental.pallas import tpu_sc as plsc

N = 10000
E = 320000
D = 128
NC = 2
NS = 16
NW = NC * NS
CHUNK = 128
NCH0 = 54
NCH1 = 108
NCHM = max(NCH0, NCH1)
EPW = (NCH0 + NCH1) * CHUNK * NS // NW
EPAD = NS * (NCH0 + NCH1) * CHUNK
NP = 10240
NACC = 10240
GR = NACC - 1
RB = 512
GRID = NP // RB
SPAN = NACC // NS

def _sc_deg_body(col_hbm, deg_out, col_v, hist_v):
    c = lax.axis_index("c")
    s = lax.axis_index("s")
    wid = s * NC + c
    zero16 = jnp.zeros((16,), jnp.float32)
    ones16 = jnp.full((16,), 1.0, jnp.float32)

    @pl.loop(0, NP // 16)
    def _zero(i):
        hist_v[pl.ds(i * 16, 16)] = zero16

    pltpu.sync_copy(col_hbm.at[pl.ds(wid * EPW, EPW)], col_v)

    @pl.loop(0, EPW // 16)
    def _hist(i):
        idx = col_v[pl.ds(i * 16, 16)]
        plsc.addupdate_scatter(hist_v, [idx], ones16)

    pltpu.sync_copy(hist_v, deg_out.at[pl.ds(wid * NP, NP)])


def _sc_agg_body(
    y_hbm, row_hbm, col_hbm, out_hbm, ri_v, ci_v, rows_v, acc, sem
):
    c = lax.axis_index("c")
    s = lax.axis_index("s")
    zero16 = jnp.zeros((16,), jnp.float32)

    @pl.loop(0, CHUNK * (D // 16))
    def _zero(i):
        rows_v[i // (D // 16), pl.ds((i % (D // 16)) * 16, 16)] = zero16

    for j in range(SPAN // CHUNK):
        pltpu.sync_copy(rows_v, acc.at[pl.ds(s * SPAN + j * CHUNK, CHUNK)])
    plsc.subcore_barrier()

    nch_me = jnp.where(c == 0, NCH0, NCH1)
    chunk0 = jnp.where(c == 0, s * NCH0, NS * NCH0 + s * NCH1)

    @pl.loop(0, nch_me)
    def _edges(i):
        base = (chunk0 + i) * CHUNK
        pltpu.sync_copy(row_hbm.at[pl.ds(base, CHUNK)], ri_v)
        pltpu.sync_copy(col_hbm.at[pl.ds(base, CHUNK)], ci_v)
        pltpu.async_copy(y_hbm.at[ri_v], rows_v, sem).wait()
        pltpu.sync_copy(rows_v, acc.at[ci_v], add=True)

    plsc.subcore_barrier()
    for j in range(SPAN // CHUNK):
        start = s * SPAN + j * CHUNK
        pltpu.sync_copy(acc.at[pl.ds(start, CHUNK)], rows_v)
        pltpu.sync_copy(rows_v, out_hbm.at[pl.ds(c * NP + start, CHUNK)])


@functools.lru_cache(maxsize=1)
def _sc_kernels():
    mesh = plsc.VectorSubcoreMesh(
        core_axis_name="c", subcore_axis_name="s",
        num_cores=NC, num_subcores=NS,
    )
    params = pltpu.CompilerParams(needs_layout_passes=False)
    sc_deg = pl.kernel(
        _sc_deg_body,
        out_type=jax.ShapeDtypeStruct((NW * NP,), jnp.float32),
        mesh=mesh,
        scratch_types=[
            pltpu.VMEM((EPW,), jnp.int32),
            pltpu.VMEM((NP,), jnp.float32),
        ],
        compiler_params=params,
    )
    sc_agg = pl.kernel(
        _sc_agg_body,
        out_type=jax.ShapeDtypeStruct((NC * NP, D), jnp.float32),
        mesh=mesh,
        scratch_types=[
            pltpu.VMEM((CHUNK,), jnp.int32),
            pltpu.VMEM((CHUNK,), jnp.int32),
            pltpu.VMEM((CHUNK, D), jnp.float32),
            pltpu.VMEM_SHARED((NACC, D), jnp.float32),
            pltpu.SemaphoreType.DMA,
        ],
        compiler_params=params,
    )
    return sc_deg, sc_agg


def _tc1_body(deg_ref, x_ref, w1_ref, y1_ref, dis_ref):
    deg = jnp.sum(deg_ref[...], axis=0) + 1.0
    dis = lax.rsqrt(deg)[:, None]
    xw = jnp.dot(x_ref[...], w1_ref[...], preferred_element_type=jnp.float32)
    y1_ref[...] = xw * dis
    dis_ref[...] = jnp.broadcast_to(dis, (RB, D))


def _tc2_body(agg_ref, y1_ref, dis_ref, w2_ref, b1_ref, g_ref, bt_ref, y2_ref):
    a = agg_ref[0] + agg_ref[1] + y1_ref[...]
    dis = dis_ref[...]
    h = dis * a + b1_ref[...]
    h = h * g_ref[...] + bt_ref[...]
    h = jnp.maximum(h, 0.0)
    y2_ref[...] = dis * jnp.dot(
        h, w2_ref[...], preferred_element_type=jnp.float32
    )


def _tc3_body(agg_ref, y2_ref, dis_ref, b2_ref, out_ref):
    out_ref[...] = (
        dis_ref[...] * (agg_ref[0] + agg_ref[1] + y2_ref[...]) + b2_ref[...]
    )


def _row_spec():
    return pl.BlockSpec((RB, D), lambda j: (j, 0))


def _full_spec():
    return pl.BlockSpec((D, D), lambda j: (0, 0))


def _vec_spec():
    return pl.BlockSpec((1, D), lambda j: (0, 0))


def _agg_spec():
    return pl.BlockSpec((NC, RB, D), lambda j: (0, j, 0))


_tc1 = pl.pallas_call(
    _tc1_body,
    grid=(GRID,),
    in_specs=[
        pl.BlockSpec((NW, RB), lambda j: (0, j)),
        _row_spec(),
        _full_spec(),
    ],
    out_specs=[_row_spec(), _row_spec()],
    out_shape=[
        jax.ShapeDtypeStruct((NP, D), jnp.float32),
        jax.ShapeDtypeStruct((NP, D), jnp.float32),
    ],
)

_tc2 = pl.pallas_call(
    _tc2_body,
    grid=(GRID,),
    in_specs=[
        _agg_spec(),
        _row_spec(),
        _row_spec(),
        _full_spec(),
        _vec_spec(),
        _vec_spec(),
        _vec_spec(),
    ],
    out_specs=_row_spec(),
    out_shape=jax.ShapeDtypeStruct((NP, D), jnp.float32),
)

_tc3 = pl.pallas_call(
    _tc3_body,
    grid=(GRID,),
    in_specs=[_agg_spec(), _row_spec(), _row_spec(), _vec_spec()],
    out_specs=_row_spec(),
    out_shape=jax.ShapeDtypeStruct((NP, D), jnp.float32),
)


@jax.jit
def kernel(x, clique_edge_index, W1, b1, gamma, beta, W2, b2):
    pad_e = EPAD - E
    row = jnp.concatenate(
        [clique_edge_index[0], jnp.zeros((pad_e,), jnp.int32)]
    )
    col = jnp.concatenate(
        [clique_edge_index[1], jnp.full((pad_e,), GR, jnp.int32)]
    )
    x_pad = jnp.zeros((NP, D), x.dtype).at[:N].set(x)

    sc_deg, sc_agg = _sc_kernels()
    deg_parts = sc_deg(col).reshape(NW, NP)
    y1, dis2 = _tc1(deg_parts, x_pad, W1)
    agg1 = sc_agg(y1, row, col).reshape(NC, NP, D)
    sg = (gamma / jnp.sqrt(1.0 + 1e-5)).reshape(1, D)
    y2 = _tc2(agg1, y1, dis2, W2, b1.reshape(1, D), sg, beta.reshape(1, D))
    agg2 = sc_agg(y2, row, col).reshape(NC, NP, D)
    out = _tc3(agg2, y2, dis2, b2.reshape(1, D))
    return out[:N]


# speedup vs baseline: 2.0135x; 2.0135x over previous
import functools

import jax
import jax.numpy as jnp
from jax import lax
from jax.experimental import pallas as pl
from jax.experimental.pallas import tpu as pltpu
from jax.experimental.pallas import tpu_sc as plsc

N = 10000
E = 320000
D = 128
NC = 2
NS = 16
NW = NC * NS
CHUNK = 128
NCH = (E + NW * CHUNK - 1) // (NW * CHUNK)
EPW = NCH * CHUNK
EPAD = EPW * NW
NP = 10240
NACC = 10240
GR = NACC - 1
RB = 512
GRID = NP // RB
SPAN = NACC // NS

def _sc_deg_body(col_hbm, deg_out, col_v, hist_v):
    c = lax.axis_index("c")
    s = lax.axis_index("s")
    wid = s * NC + c
    zero16 = jnp.zeros((16,), jnp.float32)
    ones16 = jnp.full((16,), 1.0, jnp.float32)

    @pl.loop(0, NP // 16)
    def _zero(i):
        hist_v[pl.ds(i * 16, 16)] = zero16

    pltpu.sync_copy(col_hbm.at[pl.ds(wid * EPW, EPW)], col_v)

    @pl.loop(0, EPW // 16)
    def _hist(i):
        idx = col_v[pl.ds(i * 16, 16)]
        plsc.addupdate_scatter(hist_v, [idx], ones16)

    pltpu.sync_copy(hist_v, deg_out.at[pl.ds(wid * NP, NP)])


def _sc_agg_body(
    y_hbm, row_hbm, col_hbm, out_hbm, ri_v, ci_v, rows_v, acc, sem
):
    c = lax.axis_index("c")
    s = lax.axis_index("s")
    zero16 = jnp.zeros((16,), jnp.float32)

    @pl.loop(0, CHUNK * (D // 16))
    def _zero(i):
        rows_v[i // (D // 16), pl.ds((i % (D // 16)) * 16, 16)] = zero16

    for j in range(SPAN // CHUNK):
        pltpu.sync_copy(rows_v, acc.at[pl.ds(s * SPAN + j * CHUNK, CHUNK)])
    plsc.subcore_barrier()

    wid = s * NC + c

    @pl.loop(0, EPW // CHUNK)
    def _edges(i):
        base = wid * EPW + i * CHUNK
        pltpu.sync_copy(row_hbm.at[pl.ds(base, CHUNK)], ri_v)
        pltpu.sync_copy(col_hbm.at[pl.ds(base, CHUNK)], ci_v)
        pltpu.async_copy(y_hbm.at[ri_v], rows_v, sem).wait()
        pltpu.sync_copy(rows_v, acc.at[ci_v], add=True)

    plsc.subcore_barrier()
    for j in range(SPAN // CHUNK):
        start = s * SPAN + j * CHUNK
        pltpu.sync_copy(acc.at[pl.ds(start, CHUNK)], rows_v)
        pltpu.sync_copy(rows_v, out_hbm.at[pl.ds(c * NP + start, CHUNK)])


@functools.lru_cache(maxsize=1)
def _sc_kernels():
    mesh = plsc.VectorSubcoreMesh(
        core_axis_name="c", subcore_axis_name="s",
        num_cores=NC, num_subcores=NS,
    )
    params = pltpu.CompilerParams(needs_layout_passes=False)
    sc_deg = pl.kernel(
        _sc_deg_body,
        out_type=jax.ShapeDtypeStruct((NW * NP,), jnp.float32),
        mesh=mesh,
        scratch_types=[
            pltpu.VMEM((EPW,), jnp.int32),
            pltpu.VMEM((NP,), jnp.float32),
        ],
        compiler_params=params,
    )
    sc_agg = pl.kernel(
        _sc_agg_body,
        out_type=jax.ShapeDtypeStruct((NC * NP, D), jnp.float32),
        mesh=mesh,
        scratch_types=[
            pltpu.VMEM((CHUNK,), jnp.int32),
            pltpu.VMEM((CHUNK,), jnp.int32),
            pltpu.VMEM((CHUNK, D), jnp.float32),
            pltpu.VMEM_SHARED((NACC, D), jnp.float32),
            pltpu.SemaphoreType.DMA,
        ],
        compiler_params=params,
    )
    return sc_deg, sc_agg


def _tc1_body(deg_ref, x_ref, w1_ref, y1_ref, dis_ref):
    deg = jnp.sum(deg_ref[...], axis=0) + 1.0
    dis = lax.rsqrt(deg)[:, None]
    xw = jnp.dot(x_ref[...], w1_ref[...], preferred_element_type=jnp.float32)
    y1_ref[...] = xw * dis
    dis_ref[...] = jnp.broadcast_to(dis, (RB, D))


def _tc2_body(agg_ref, y1_ref, dis_ref, w2_ref, b1_ref, g_ref, bt_ref, y2_ref):
    a = agg_ref[0] + agg_ref[1] + y1_ref[...]
    dis = dis_ref[...]
    h = dis * a + b1_ref[...]
    h = h * g_ref[...] + bt_ref[...]
    h = jnp.maximum(h, 0.0)
    y2_ref[...] = dis * jnp.dot(
        h, w2_ref[...], preferred_element_type=jnp.float32
    )


def _tc3_body(agg_ref, y2_ref, dis_ref, b2_ref, out_ref):
    out_ref[...] = (
        dis_ref[...] * (agg_ref[0] + agg_ref[1] + y2_ref[...]) + b2_ref[...]
    )


def _row_spec():
    return pl.BlockSpec((RB, D), lambda j: (j, 0))


def _full_spec():
    return pl.BlockSpec((D, D), lambda j: (0, 0))


def _vec_spec():
    return pl.BlockSpec((1, D), lambda j: (0, 0))


def _agg_spec():
    return pl.BlockSpec((NC, RB, D), lambda j: (0, j, 0))


_tc1 = pl.pallas_call(
    _tc1_body,
    grid=(GRID,),
    in_specs=[
        pl.BlockSpec((NW, RB), lambda j: (0, j)),
        _row_spec(),
        _full_spec(),
    ],
    out_specs=[_row_spec(), _row_spec()],
    out_shape=[
        jax.ShapeDtypeStruct((NP, D), jnp.float32),
        jax.ShapeDtypeStruct((NP, D), jnp.float32),
    ],
)

_tc2 = pl.pallas_call(
    _tc2_body,
    grid=(GRID,),
    in_specs=[
        _agg_spec(),
        _row_spec(),
        _row_spec(),
        _full_spec(),
        _vec_spec(),
        _vec_spec(),
        _vec_spec(),
    ],
    out_specs=_row_spec(),
    out_shape=jax.ShapeDtypeStruct((NP, D), jnp.float32),
)

_tc3 = pl.pallas_call(
    _tc3_body,
    grid=(GRID,),
    in_specs=[_agg_spec(), _row_spec(), _row_spec(), _vec_spec()],
    out_specs=_row_spec(),
    out_shape=jax.ShapeDtypeStruct((NP, D), jnp.float32),
)


@jax.jit
def kernel(x, clique_edge_index, W1, b1, gamma, beta, W2, b2):
    pad_e = EPAD - E
    row = jnp.concatenate(
        [clique_edge_index[0], jnp.zeros((pad_e,), jnp.int32)]
    )
    col = jnp.concatenate(
        [clique_edge_index[1], jnp.full((pad_e,), GR, jnp.int32)]
    )
    x_pad = jnp.zeros((NP, D), x.dtype).at[:N].set(x)

    sc_deg, sc_agg = _sc_kernels()
    deg_parts = sc_deg(col).reshape(NW, NP)
    y1, dis2 = _tc1(deg_parts, x_pad, W1)
    agg1 = sc_agg(y1, row, col).reshape(NC, NP, D)
    sg = (gamma / jnp.sqrt(1.0 + 1e-5)).reshape(1, D)
    y2 = _tc2(agg1, y1, dis2, W2, b1.reshape(1, D), sg, beta.reshape(1, D))
    agg2 = sc_agg(y2, row, col).reshape(NC, NP, D)
    out = _tc3(agg2, y2, dis2, b2.reshape(1, D))
    return out[:N]
